# baseline (speedup 1.0000x reference)
import functools

import jax
import jax.numpy as jnp
from jax.experimental import pallas as pl
from jax.experimental.pallas import tpu as pltpu

_NUM_SEGMENTS = 32
_SEG_LENGTH = 16
_K = 3
_BN_EPS = 1e-5
_B = 64
_T = _NUM_SEGMENTS * _SEG_LENGTH
_D = 768
_C = 100
_CP = 128
_N = _B * _T


def _p1_body(text_ref, nc_ref, img_ref, raw_ref, stats_ref, txt_ref, acc_ref):
    i = pl.program_id(0)

    @pl.when(i == 0)
    def _init():
        txt = text_ref[1:_C + 1, :] - nc_ref[0:1, :]
        nrm = jnp.sqrt(jnp.sum(txt * txt, axis=1, keepdims=True))
        txt_ref[0:_C, :] = txt / nrm
        txt_ref[_C:_CP, :] = jnp.zeros((_CP - _C, _D), jnp.float32)
        acc_ref[...] = jnp.zeros_like(acc_ref)

    x = img_ref[...] - nc_ref[0:1, :]
    t = txt_ref[...]
    y = jax.lax.dot_general(x, t, (((1,), (1,)), ((), ())),
                            preferred_element_type=jnp.float32)
    raw_ref[...] = y
    acc_ref[0:1, :] += jnp.sum(y, axis=0, keepdims=True)
    acc_ref[1:2, :] += jnp.sum(y * y, axis=0, keepdims=True)

    @pl.when(i == pl.num_programs(0) - 1)
    def _fin():
        n = jnp.float32(_N)
        mean = acc_ref[0:1, :] / n
        var = acc_ref[1:2, :] / n - mean * mean
        stats_ref[0:1, :] = mean
        stats_ref[1:2, :] = 1.0 / jnp.sqrt(var + _BN_EPS)
        stats_ref[2:8, :] = jnp.zeros((6, _CP), jnp.float32)


_VPB = 8
_SPB = _VPB * _NUM_SEGMENTS


def _p2_body(lab_ref, raw_ref, stats_ref,
             norm_ref, topk_ref, botk_ref, tidx_ref, bidx_ref):
    b = pl.program_id(0)
    mean = stats_ref[0:1, :]
    inv = stats_ref[1:2, :]
    y = (raw_ref[...] - mean) * inv
    norm_ref[...] = y[:, :_C]

    segsum = jnp.sum(y.reshape(_SPB, _SEG_LENGTH, _CP), axis=1)

    ridx = jax.lax.broadcasted_iota(jnp.int32, (_SPB, 1), 0)
    vid = ridx // _NUM_SEGMENTS
    labs = [lab_ref[b * _VPB + j] for j in range(_VPB)]
    alabs = [lab - (lab > 0).astype(lab.dtype) for lab in labs]
    lrow = alabs[_VPB - 1]
    for j in range(_VPB - 2, -1, -1):
        lrow = jnp.where(vid == j, alabs[j], lrow)
    lane = jax.lax.broadcasted_iota(jnp.int32, (_SPB, _CP), 1)
    onehot = (lane == lrow).astype(jnp.float32)
    w = jnp.where(b < (_B // 2) // _VPB, onehot, jnp.ones_like(onehot))
    key = jnp.sum(segsum * w, axis=1, keepdims=True)

    li = jax.lax.broadcasted_iota(jnp.int32, (1, 1, _CP), 2)

    for j in range(_VPB):
        in_vid = vid == j

        def _pick3(largest):
            vals = jnp.where(in_vid, key,
                             jnp.float32(-1e30) if largest else jnp.float32(1e30))
            picks = []
            for _ in range(_K):
                m = jnp.max(vals) if largest else jnp.min(vals)
                i_k = jnp.min(jnp.where(vals == m, ridx, _SPB))
                picks.append(i_k)
                fill = jnp.float32(-1e30) if largest else jnp.float32(1e30)
                vals = jnp.where(ridx == i_k, fill, vals)
            return picks

        tis = _pick3(True)
        bis = _pick3(False)

        for k in range(_K):
            ts = pl.multiple_of(tis[k] * _SEG_LENGTH, _SEG_LENGTH)
            bs = pl.multiple_of(bis[k] * _SEG_LENGTH, _SEG_LENGTH)
            r0 = (j * _K + k) * _SEG_LENGTH
            topk_ref[r0:r0 + _SEG_LENGTH, :] = norm_ref[pl.ds(ts, _SEG_LENGTH), :]
            botk_ref[r0:r0 + _SEG_LENGTH, :] = norm_ref[pl.ds(bs, _SEG_LENGTH), :]

        def _idxvec(picks):
            base = j * _NUM_SEGMENTS
            return jnp.where(li == 0, picks[0] - base,
                             jnp.where(li == 1, picks[1] - base,
                                       jnp.where(li == 2, picks[2] - base, 0)))

        tidx_ref[j:j + 1] = _idxvec(tis)
        bidx_ref[j:j + 1] = _idxvec(bis)


@functools.partial(jax.jit, static_argnames=("interpret",))
def _run(image_features, text_features, labels, ncentroid, interpret=False):
    img = image_features.reshape(_N, _D)
    nc = ncentroid.reshape(1, _D)

    p1_rows = 2048
    raw, stats = pl.pallas_call(
        _p1_body,
        grid=(_N // p1_rows,),
        in_specs=[
            pl.BlockSpec((_C + 1, _D), lambda i: (0, 0)),
            pl.BlockSpec((1, _D), lambda i: (0, 0)),
            pl.BlockSpec((p1_rows, _D), lambda i: (i, 0)),
        ],
        out_specs=[
            pl.BlockSpec((p1_rows, _CP), lambda i: (i, 0)),
            pl.BlockSpec((8, _CP), lambda i: (0, 0)),
        ],
        out_shape=[
            jax.ShapeDtypeStruct((_N, _CP), jnp.float32),
            jax.ShapeDtypeStruct((8, _CP), jnp.float32),
        ],
        scratch_shapes=[
            pltpu.VMEM((_CP, _D), jnp.float32),
            pltpu.VMEM((8, _CP), jnp.float32),
        ],
        interpret=interpret,
    )(text_features, nc, img)

    grid_spec = pltpu.PrefetchScalarGridSpec(
        num_scalar_prefetch=1,
        grid=(_B // _VPB,),
        in_specs=[
            pl.BlockSpec((_VPB * _T, _CP), lambda b, lab: (b, 0)),
            pl.BlockSpec((8, _CP), lambda b, lab: (0, 0)),
        ],
        out_specs=[
            pl.BlockSpec((_VPB * _T, _C), lambda b, lab: (b, 0)),
            pl.BlockSpec((_VPB * _K * _SEG_LENGTH, _C), lambda b, lab: (b, 0)),
            pl.BlockSpec((_VPB * _K * _SEG_LENGTH, _C), lambda b, lab: (b, 0)),
            pl.BlockSpec((_VPB, 1, _CP), lambda b, lab: (b, 0, 0)),
            pl.BlockSpec((_VPB, 1, _CP), lambda b, lab: (b, 0, 0)),
        ],
    )
    norm, topk, botk, tidx, bidx = pl.pallas_call(
        _p2_body,
        grid_spec=grid_spec,
        out_shape=[
            jax.ShapeDtypeStruct((_N, _C), jnp.float32),
            jax.ShapeDtypeStruct((_B * _K * _SEG_LENGTH, _C), jnp.float32),
            jax.ShapeDtypeStruct((_B * _K * _SEG_LENGTH, _C), jnp.float32),
            jax.ShapeDtypeStruct((_B, 1, _CP), jnp.int32),
            jax.ShapeDtypeStruct((_B, 1, _CP), jnp.int32),
        ],
        interpret=interpret,
    )(labels.astype(jnp.int32), raw, stats)

    return (norm, topk, botk,
            tidx[:_B // 2, 0, :_K], tidx[_B // 2:, 0, :_K],
            bidx[:_B // 2, 0, :_K])


def kernel(image_features, text_features, labels, ncentroid, test_mode):
    return _run(image_features, text_features, labels, ncentroid)


# speedup vs baseline: 1.0853x; 1.0853x over previous
import functools

import jax
import jax.numpy as jnp
from jax.experimental import pallas as pl
from jax.experimental.pallas import tpu as pltpu

_NUM_SEGMENTS = 32
_SEG_LENGTH = 16
_K = 3
_BN_EPS = 1e-5
_B = 64
_T = _NUM_SEGMENTS * _SEG_LENGTH
_D = 768
_C = 100
_CP = 128
_N = _B * _T

_VPB = 4
_ROWS = _VPB * _T
_NBLK = _N // _ROWS
_SPB = _VPB * _NUM_SEGMENTS


def _body(lab_ref, text_ref, nc_ref, img_ref,
          norm_ref, topk_ref, botk_ref, tidx_ref, bidx_ref,
          raw_ref, txt_ref, acc_ref):
    t = pl.program_id(0)
    i = pl.program_id(1)

    @pl.when((t == 0) & (i == 0))
    def _init():
        txt = text_ref[1:_C + 1, :] - nc_ref[0:1, :]
        nrm = jnp.sqrt(jnp.sum(txt * txt, axis=1, keepdims=True))
        txt_ref[0:_C, :] = txt / nrm
        txt_ref[_C:_CP, :] = jnp.zeros((_CP - _C, _D), jnp.float32)
        acc_ref[...] = jnp.zeros_like(acc_ref)

    @pl.when(t == 0)
    def _matmul():
        x = img_ref[...] - nc_ref[0:1, :]
        y = jax.lax.dot_general(x, txt_ref[...], (((1,), (1,)), ((), ())),
                                preferred_element_type=jnp.float32)
        raw_ref[pl.ds(i * _ROWS, _ROWS), :] = y
        acc_ref[0:1, :] += jnp.sum(y, axis=0, keepdims=True)
        acc_ref[1:2, :] += jnp.sum(y * y, axis=0, keepdims=True)

        @pl.when(i == _NBLK - 1)
        def _fin():
            n = jnp.float32(_N)
            mean = acc_ref[0:1, :] / n
            var = acc_ref[1:2, :] / n - mean * mean
            acc_ref[2:3, :] = mean
            acc_ref[3:4, :] = 1.0 / jnp.sqrt(var + _BN_EPS)

    @pl.when(t == 1)
    def _select():
        mean = acc_ref[2:3, :]
        inv = acc_ref[3:4, :]
        y = (raw_ref[pl.ds(i * _ROWS, _ROWS), :] - mean) * inv
        norm_ref[...] = y[:, :_C]

        segsum = jnp.sum(y.reshape(_SPB, _SEG_LENGTH, _CP), axis=1)

        ridx = jax.lax.broadcasted_iota(jnp.int32, (_SPB, 1), 0)
        vid = ridx // _NUM_SEGMENTS
        labs = [lab_ref[i * _VPB + j] for j in range(_VPB)]
        alabs = [lab - (lab > 0).astype(lab.dtype) for lab in labs]
        lrow = alabs[_VPB - 1]
        for j in range(_VPB - 2, -1, -1):
            lrow = jnp.where(vid == j, alabs[j], lrow)
        lane = jax.lax.broadcasted_iota(jnp.int32, (_SPB, _CP), 1)
        onehot = (lane == lrow).astype(jnp.float32)
        w = jnp.where(i < (_B // 2) // _VPB, onehot, jnp.ones_like(onehot))
        key = jnp.sum(segsum * w, axis=1, keepdims=True)

        li = jax.lax.broadcasted_iota(jnp.int32, (1, 1, _CP), 2)

        for j in range(_VPB):
            in_vid = vid == j

            def _pick3(largest):
                vals = jnp.where(
                    in_vid, key,
                    jnp.float32(-1e30) if largest else jnp.float32(1e30))
                picks = []
                for _ in range(_K):
                    m = jnp.max(vals) if largest else jnp.min(vals)
                    i_k = jnp.min(jnp.where(vals == m, ridx, _SPB))
                    picks.append(i_k)
                    fill = jnp.float32(-1e30) if largest else jnp.float32(1e30)
                    vals = jnp.where(ridx == i_k, fill, vals)
                return picks

            tis = _pick3(True)
            bis = _pick3(False)

            for k in range(_K):
                ts = pl.multiple_of(tis[k] * _SEG_LENGTH, _SEG_LENGTH)
                bs = pl.multiple_of(bis[k] * _SEG_LENGTH, _SEG_LENGTH)
                r0 = (j * _K + k) * _SEG_LENGTH
                topk_ref[r0:r0 + _SEG_LENGTH, :] = \
                    norm_ref[pl.ds(ts, _SEG_LENGTH), :]
                botk_ref[r0:r0 + _SEG_LENGTH, :] = \
                    norm_ref[pl.ds(bs, _SEG_LENGTH), :]

            def _idxvec(picks):
                base = j * _NUM_SEGMENTS
                return jnp.where(li == 0, picks[0] - base,
                                 jnp.where(li == 1, picks[1] - base,
                                           jnp.where(li == 2, picks[2] - base,
                                                     0)))

            tidx_ref[j:j + 1] = _idxvec(tis)
            bidx_ref[j:j + 1] = _idxvec(bis)


@functools.partial(jax.jit, static_argnames=("interpret",))
def _run(image_features, text_features, labels, ncentroid, interpret=False):
    img = image_features.reshape(_N, _D)
    nc = ncentroid.reshape(1, _D)

    grid_spec = pltpu.PrefetchScalarGridSpec(
        num_scalar_prefetch=1,
        grid=(2, _NBLK),
        in_specs=[
            pl.BlockSpec((_C + 1, _D), lambda t, i, lab: (0, 0)),
            pl.BlockSpec((1, _D), lambda t, i, lab: (0, 0)),
            pl.BlockSpec((_ROWS, _D),
                         lambda t, i, lab: (jnp.where(t == 0, i, _NBLK - 1), 0)),
        ],
        out_specs=[
            pl.BlockSpec((_ROWS, _C),
                         lambda t, i, lab: (jnp.where(t == 1, i, _NBLK - 1), 0)),
            pl.BlockSpec((_VPB * _K * _SEG_LENGTH, _C),
                         lambda t, i, lab: (jnp.where(t == 1, i, _NBLK - 1), 0)),
            pl.BlockSpec((_VPB * _K * _SEG_LENGTH, _C),
                         lambda t, i, lab: (jnp.where(t == 1, i, _NBLK - 1), 0)),
            pl.BlockSpec((_VPB, 1, _CP),
                         lambda t, i, lab: (jnp.where(t == 1, i, _NBLK - 1), 0, 0)),
            pl.BlockSpec((_VPB, 1, _CP),
                         lambda t, i, lab: (jnp.where(t == 1, i, _NBLK - 1), 0, 0)),
        ],
        scratch_shapes=[
            pltpu.VMEM((_N, _CP), jnp.float32),
            pltpu.VMEM((_CP, _D), jnp.float32),
            pltpu.VMEM((8, _CP), jnp.float32),
        ],
    )
    norm, topk, botk, tidx, bidx = pl.pallas_call(
        _body,
        grid_spec=grid_spec,
        out_shape=[
            jax.ShapeDtypeStruct((_N, _C), jnp.float32),
            jax.ShapeDtypeStruct((_B * _K * _SEG_LENGTH, _C), jnp.float32),
            jax.ShapeDtypeStruct((_B * _K * _SEG_LENGTH, _C), jnp.float32),
            jax.ShapeDtypeStruct((_B, 1, _CP), jnp.int32),
            jax.ShapeDtypeStruct((_B, 1, _CP), jnp.int32),
        ],
        interpret=interpret,
    )(labels.astype(jnp.int32), text_features, nc, img)

    return (norm, topk, botk,
            tidx[:_B // 2, 0, :_K], tidx[_B // 2:, 0, :_K],
            bidx[:_B // 2, 0, :_K])


def kernel(image_features, text_features, labels, ncentroid, test_mode):
    return _run(image_features, text_features, labels, ncentroid)


# speedup vs baseline: 1.0950x; 1.0089x over previous
import functools

import jax
import jax.numpy as jnp
from jax.experimental import pallas as pl
from jax.experimental.pallas import tpu as pltpu

_NUM_SEGMENTS = 32
_SEG_LENGTH = 16
_K = 3
_BN_EPS = 1e-5
_B = 64
_T = _NUM_SEGMENTS * _SEG_LENGTH
_D = 768
_C = 100
_CP = 128
_N = _B * _T

_VPB = 4
_ROWS = _VPB * _T
_NBLK = _N // _ROWS
_SPB = _VPB * _NUM_SEGMENTS


def _body(lab_ref, text_ref, nc_ref, img_ref,
          norm_ref, topk_ref, botk_ref, ta_ref, tn_ref, ba_ref,
          raw_ref, txt_ref, acc_ref, idx_ref):
    t = pl.program_id(0)
    i = pl.program_id(1)

    @pl.when((t == 0) & (i == 0))
    def _init():
        txt = text_ref[1:_C + 1, :] - nc_ref[0:1, :]
        nrm = jnp.sqrt(jnp.sum(txt * txt, axis=1, keepdims=True))
        txt_ref[0:_C, :] = txt / nrm
        txt_ref[_C:_CP, :] = jnp.zeros((_CP - _C, _D), jnp.float32)
        acc_ref[...] = jnp.zeros_like(acc_ref)

    @pl.when(t == 0)
    def _matmul():
        x = img_ref[...] - nc_ref[0:1, :]
        y = jax.lax.dot_general(x, txt_ref[...], (((1,), (1,)), ((), ())),
                                preferred_element_type=jnp.float32)
        raw_ref[pl.ds(i * _ROWS, _ROWS), :] = y
        acc_ref[0:1, :] += jnp.sum(y, axis=0, keepdims=True)
        acc_ref[1:2, :] += jnp.sum(y * y, axis=0, keepdims=True)

        @pl.when(i == _NBLK - 1)
        def _fin():
            n = jnp.float32(_N)
            mean = acc_ref[0:1, :] / n
            var = acc_ref[1:2, :] / n - mean * mean
            acc_ref[2:3, :] = mean
            acc_ref[3:4, :] = 1.0 / jnp.sqrt(var + _BN_EPS)

    @pl.when(t == 1)
    def _select():
        mean = acc_ref[2:3, :]
        inv = acc_ref[3:4, :]
        y = (raw_ref[pl.ds(i * _ROWS, _ROWS), :] - mean) * inv
        norm_ref[...] = y[:, :_C]

        segsum = jnp.sum(y.reshape(_SPB, _SEG_LENGTH, _CP), axis=1)

        ridx = jax.lax.broadcasted_iota(jnp.int32, (_SPB, 1), 0)
        vid = ridx // _NUM_SEGMENTS
        labs = [lab_ref[i * _VPB + j] for j in range(_VPB)]
        alabs = [lab - (lab > 0).astype(lab.dtype) for lab in labs]
        lrow = alabs[_VPB - 1]
        for j in range(_VPB - 2, -1, -1):
            lrow = jnp.where(vid == j, alabs[j], lrow)
        lane = jax.lax.broadcasted_iota(jnp.int32, (_SPB, _CP), 1)
        onehot = (lane == lrow).astype(jnp.float32)
        w = jnp.where(i < (_B // 2) // _VPB, onehot, jnp.ones_like(onehot))
        key = jnp.sum(segsum * w, axis=1, keepdims=True)

        li = jax.lax.broadcasted_iota(jnp.int32, (1, _CP), 1)

        for j in range(_VPB):
            in_vid = vid == j

            def _pick3(largest):
                vals = jnp.where(
                    in_vid, key,
                    jnp.float32(-1e30) if largest else jnp.float32(1e30))
                picks = []
                for _ in range(_K):
                    m = jnp.max(vals) if largest else jnp.min(vals)
                    i_k = jnp.min(jnp.where(vals == m, ridx, _SPB))
                    picks.append(i_k)
                    fill = jnp.float32(-1e30) if largest else jnp.float32(1e30)
                    vals = jnp.where(ridx == i_k, fill, vals)
                return picks

            tis = _pick3(True)
            bis = _pick3(False)

            for k in range(_K):
                ts = pl.multiple_of(tis[k] * _SEG_LENGTH, _SEG_LENGTH)
                bs = pl.multiple_of(bis[k] * _SEG_LENGTH, _SEG_LENGTH)
                r0 = (j * _K + k) * _SEG_LENGTH
                topk_ref[r0:r0 + _SEG_LENGTH, :] = \
                    norm_ref[pl.ds(ts, _SEG_LENGTH), :]
                botk_ref[r0:r0 + _SEG_LENGTH, :] = \
                    norm_ref[pl.ds(bs, _SEG_LENGTH), :]

            def _idxvec(picks):
                base = j * _NUM_SEGMENTS
                return jnp.where(li == 0, picks[0] - base,
                                 jnp.where(li == 1, picks[1] - base,
                                           jnp.where(li == 2, picks[2] - base,
                                                     0)))

            v = i * _VPB + j
            idx_ref[pl.ds(v, 1), :] = _idxvec(tis)
            idx_ref[pl.ds(_B + v, 1), :] = _idxvec(bis)

        @pl.when(i == _NBLK - 1)
        def _emit_idx():
            ta_ref[...] = idx_ref[0:_B // 2, 0:_K]
            tn_ref[...] = idx_ref[_B // 2:_B, 0:_K]
            ba_ref[...] = idx_ref[_B:_B + _B // 2, 0:_K]


@functools.partial(jax.jit, static_argnames=("interpret",))
def _run(image_features, text_features, labels, ncentroid, interpret=False):
    img = image_features.reshape(_N, _D)
    nc = ncentroid.reshape(1, _D)

    grid_spec = pltpu.PrefetchScalarGridSpec(
        num_scalar_prefetch=1,
        grid=(2, _NBLK),
        in_specs=[
            pl.BlockSpec((_C + 1, _D), lambda t, i, lab: (0, 0)),
            pl.BlockSpec((1, _D), lambda t, i, lab: (0, 0)),
            pl.BlockSpec((_ROWS, _D),
                         lambda t, i, lab: (jnp.where(t == 0, i, _NBLK - 1), 0)),
        ],
        out_specs=[
            pl.BlockSpec((_ROWS, _C),
                         lambda t, i, lab: (jnp.where(t == 1, i, _NBLK - 1), 0)),
            pl.BlockSpec((_VPB * _K * _SEG_LENGTH, _C),
                         lambda t, i, lab: (jnp.where(t == 1, i, _NBLK - 1), 0)),
            pl.BlockSpec((_VPB * _K * _SEG_LENGTH, _C),
                         lambda t, i, lab: (jnp.where(t == 1, i, _NBLK - 1), 0)),
            pl.BlockSpec((_B // 2, _K), lambda t, i, lab: (0, 0)),
            pl.BlockSpec((_B // 2, _K), lambda t, i, lab: (0, 0)),
            pl.BlockSpec((_B // 2, _K), lambda t, i, lab: (0, 0)),
        ],
        scratch_shapes=[
            pltpu.VMEM((_N, _CP), jnp.float32),
            pltpu.VMEM((_CP, _D), jnp.float32),
            pltpu.VMEM((8, _CP), jnp.float32),
            pltpu.VMEM((2 * _B, _CP), jnp.int32),
        ],
    )
    norm, topk, botk, ta, tn, ba = pl.pallas_call(
        _body,
        grid_spec=grid_spec,
        out_shape=[
            jax.ShapeDtypeStruct((_N, _C), jnp.float32),
            jax.ShapeDtypeStruct((_B * _K * _SEG_LENGTH, _C), jnp.float32),
            jax.ShapeDtypeStruct((_B * _K * _SEG_LENGTH, _C), jnp.float32),
            jax.ShapeDtypeStruct((_B // 2, _K), jnp.int32),
            jax.ShapeDtypeStruct((_B // 2, _K), jnp.int32),
            jax.ShapeDtypeStruct((_B // 2, _K), jnp.int32),
        ],
        interpret=interpret,
    )(labels.astype(jnp.int32), text_features, nc, img)

    return (norm, topk, botk, ta, tn, ba)


def kernel(image_features, text_features, labels, ncentroid, test_mode):
    return _run(image_features, text_features, labels, ncentroid)
